# Initial kernel scaffold; baseline (speedup 1.0000x reference)
import jax
import jax.numpy as jnp
from jax.experimental import pallas as pl


def kernel(x, conv1_w, conv1_gamma, conv1_beta, conv2_w, conv2_gamma, conv2_beta, conv3_w, conv3_gamma, conv3_beta, conv4_w, conv4_gamma, conv4_beta, conv5_w, conv5_gamma, conv5_beta):
    raise NotImplementedError("write your pallas kernel here")


# speedup vs baseline: 1.5789x; 1.5789x over previous
import functools

import jax
import jax.numpy as jnp
from jax.experimental import pallas as pl
from jax.experimental.pallas import tpu as pltpu

BN_EPS = 1e-5
NEG_SLOPE = 0.2
VMEM_LIMIT = 64 * 1024 * 1024
NEG_BIG = -3.0e38


def _lrelu(y):
    return jnp.where(y > 0, y, NEG_SLOPE * y)


def _edge_kernel(raw_ref, sc_ref, sh_ref, w_ref, out_ref, st_ref,
                 *, K, N, act):
    r = raw_ref[0]
    if act:
        y0 = r * sc_ref[...] + sh_ref[...]
        xa = _lrelu(y0)
    else:
        xa = r

    g = jax.lax.dot_general(xa, xa, (((1,), (1,)), ((), ())),
                            preferred_element_type=jnp.float32)
    xsq = xa * xa
    sq_col = jnp.sum(xsq, axis=1, keepdims=True)
    ones_row = jnp.ones((1, xa.shape[-1]), xa.dtype)
    sq_row = jax.lax.dot_general(ones_row, xsq, (((1,), (1,)), ((), ())),
                                 preferred_element_type=jnp.float32)
    d = 2.0 * g - sq_col - sq_row

    iota = jax.lax.broadcasted_iota(jnp.int32, (N, N), 1)
    onehots = []
    for _ in range(K):
        m = jnp.max(d, axis=1, keepdims=True)
        idx = jnp.min(jnp.where(d == m, iota, N), axis=1,
                      keepdims=True)
        oh = iota == idx
        onehots.append(oh.astype(jnp.bfloat16))
        d = jnp.where(oh, NEG_BIG, d)
    onehot = jnp.concatenate(onehots, axis=0)

    x_bf = xa.astype(jnp.bfloat16)
    cp = x_bf.shape[-1]
    nbr = jax.lax.dot_general(
        onehot, x_bf, (((1,), (0,)), ((), ())),
        preferred_element_type=jnp.float32).astype(jnp.bfloat16)
    ctr = jnp.broadcast_to(x_bf[None], (K, N, cp)).reshape(K * N, cp)
    feat = jnp.concatenate([nbr - ctr, ctr], axis=-1)
    y2 = jnp.dot(feat, w_ref[...], preferred_element_type=jnp.float32)
    cout = y2.shape[-1]
    s = jnp.sum(y2, axis=0, keepdims=True)
    ss = jnp.sum(y2 * y2, axis=0, keepdims=True)
    st_ref[0] = jnp.concatenate([s, ss], axis=0)

    out_ref[0] = jnp.max(y2.reshape(K, N, cout), axis=0)


def _edge_block(raw, sc, sh, w, K, act):
    B, N, cp = raw.shape
    cout = w.shape[1]
    body = functools.partial(_edge_kernel, K=K, N=N, act=act)
    return pl.pallas_call(
        body,
        grid=(B,),
        in_specs=[
            pl.BlockSpec((1, N, cp), lambda b: (b, 0, 0)),
            pl.BlockSpec((1, cp), lambda b: (0, 0)),
            pl.BlockSpec((1, cp), lambda b: (0, 0)),
            pl.BlockSpec((2 * cp, cout), lambda b: (0, 0)),
        ],
        out_specs=(
            pl.BlockSpec((1, N, cout), lambda b: (b, 0, 0)),
            pl.BlockSpec((1, 2, cout), lambda b: (b, 0, 0)),
        ),
        out_shape=(
            jax.ShapeDtypeStruct((B, N, cout), jnp.float32),
            jax.ShapeDtypeStruct((B, 2, cout), jnp.float32),
        ),
        compiler_params=pltpu.CompilerParams(
            dimension_semantics=("parallel",),
            vmem_limit_bytes=VMEM_LIMIT),
    )(raw, sc, sh, w)


def _bn_scale_shift(stats, count, gamma, beta):
    s = stats.sum(axis=0)
    mean = s[0] / count
    var = jnp.maximum(s[1] / count - mean * mean, 0.0)
    inv = jax.lax.rsqrt(var + BN_EPS)
    scale = gamma.reshape(-1) * inv
    shift = beta.reshape(-1) - mean * scale
    return scale.reshape(1, -1), shift.reshape(1, -1)


def _gram_kernel(r1_ref, r2_ref, r3_ref, r4_ref, sc_ref, sh_ref,
                 g_ref, s_ref):
    j = pl.program_id(1)
    cat_raw = jnp.concatenate(
        [r1_ref[...], r2_ref[...], r3_ref[...], r4_ref[...]], axis=-1)
    a = _lrelu(cat_raw * sc_ref[...] + sh_ref[...])
    abf = a.astype(jnp.bfloat16)
    g = jax.lax.dot_general(abf, abf, (((0,), (0,)), ((), ())),
                            preferred_element_type=jnp.float32)
    s = jnp.sum(abf.astype(jnp.float32), axis=0, keepdims=True)

    @pl.when(j == 0)
    def _():
        g_ref[0] = g
        s_ref[0] = s

    @pl.when(j != 0)
    def _():
        g_ref[0] += g
        s_ref[0] += s


def _final_kernel(r1_ref, r2_ref, r3_ref, r4_ref, sc_ref, sh_ref,
                  w5_ref, s5_ref, h5_ref, out_ref):
    cat_raw = jnp.concatenate(
        [r1_ref[0], r2_ref[0], r3_ref[0], r4_ref[0]], axis=-1)
    a = _lrelu(cat_raw * sc_ref[...] + sh_ref[...])
    abf = a.astype(jnp.bfloat16)
    y = jnp.dot(abf, w5_ref[...], preferred_element_type=jnp.float32)
    z = _lrelu(y * s5_ref[...] + h5_ref[...])
    out_ref[0] = jnp.transpose(z, (1, 0))


def _prep_w(w, cin, cp):
    cout = w.shape[1]
    wp = jnp.zeros((2 * cp, cout), jnp.float32)
    wp = wp.at[:cin].set(w[:cin])
    wp = wp.at[cp:cp + cin].set(w[cin:])
    return wp.astype(jnp.bfloat16)


def kernel(x, conv1_w, conv1_gamma, conv1_beta,
           conv2_w, conv2_gamma, conv2_beta,
           conv3_w, conv3_gamma, conv3_beta,
           conv4_w, conv4_gamma, conv4_beta,
           conv5_w, conv5_gamma, conv5_beta):
    K = 20
    B, _, N = x.shape
    xT = jnp.transpose(x, (0, 2, 1))
    xp = jnp.pad(xT, ((0, 0), (0, 0), (0, 5)))

    w1 = _prep_w(conv1_w, 3, 8)
    w2 = _prep_w(conv2_w, 64, 64)
    w3 = _prep_w(conv3_w, 64, 64)
    w4 = _prep_w(conv4_w, 128, 128)

    zero8 = jnp.zeros((1, 8), jnp.float32)
    cnt = float(B * N * K)

    raw1, st1 = _edge_block(xp, zero8, zero8, w1, K, act=False)
    sc1, sh1 = _bn_scale_shift(st1, cnt, conv1_gamma, conv1_beta)
    raw2, st2 = _edge_block(raw1, sc1, sh1, w2, K, act=True)
    sc2, sh2 = _bn_scale_shift(st2, cnt, conv2_gamma, conv2_beta)
    raw3, st3 = _edge_block(raw2, sc2, sh2, w3, K, act=True)
    sc3, sh3 = _bn_scale_shift(st3, cnt, conv3_gamma, conv3_beta)
    raw4, st4 = _edge_block(raw3, sc3, sh3, w4, K, act=True)
    sc4, sh4 = _bn_scale_shift(st4, cnt, conv4_gamma, conv4_beta)

    sc_cat = jnp.concatenate([sc1, sc2, sc3, sc4], axis=-1)
    sh_cat = jnp.concatenate([sh1, sh2, sh3, sh4], axis=-1)

    rows = B * N
    f1 = raw1.reshape(rows, 64)
    f2 = raw2.reshape(rows, 64)
    f3 = raw3.reshape(rows, 128)
    f4 = raw4.reshape(rows, 256)
    TR = min(2048, rows // 2)
    while TR > 8 and rows % (2 * TR):
        TR -= 8
    nt = rows // (2 * TR)
    gpart, spart = pl.pallas_call(
        _gram_kernel,
        grid=(2, nt),
        in_specs=[
            pl.BlockSpec((TR, 64), lambda c, j: (c * nt + j, 0)),
            pl.BlockSpec((TR, 64), lambda c, j: (c * nt + j, 0)),
            pl.BlockSpec((TR, 128), lambda c, j: (c * nt + j, 0)),
            pl.BlockSpec((TR, 256), lambda c, j: (c * nt + j, 0)),
            pl.BlockSpec((1, 512), lambda c, j: (0, 0)),
            pl.BlockSpec((1, 512), lambda c, j: (0, 0)),
        ],
        out_specs=(
            pl.BlockSpec((1, 512, 512), lambda c, j: (c, 0, 0)),
            pl.BlockSpec((1, 1, 512), lambda c, j: (c, 0, 0)),
        ),
        out_shape=(
            jax.ShapeDtypeStruct((2, 512, 512), jnp.float32),
            jax.ShapeDtypeStruct((2, 1, 512), jnp.float32),
        ),
        compiler_params=pltpu.CompilerParams(
            dimension_semantics=("parallel", "arbitrary"),
            vmem_limit_bytes=VMEM_LIMIT),
    )(f1, f2, f3, f4, sc_cat, sh_cat)

    w5bf = conv5_w.astype(jnp.bfloat16)
    w5f = w5bf.astype(jnp.float32)
    gsum = gpart.sum(axis=0)
    ssum = spart.sum(axis=0).reshape(512)
    mean5 = (ssum @ w5f) / rows
    ey2 = jnp.sum(w5f * (gsum @ w5f), axis=0) / rows
    var5 = jnp.maximum(ey2 - mean5 * mean5, 0.0)
    inv5 = jax.lax.rsqrt(var5 + BN_EPS)
    scale5 = (conv5_gamma.reshape(-1) * inv5).reshape(1, -1)
    shift5 = (conv5_beta.reshape(-1) - mean5 * scale5.reshape(-1)).reshape(1, -1)

    emb = conv5_w.shape[1]
    out = pl.pallas_call(
        _final_kernel,
        grid=(B,),
        in_specs=[
            pl.BlockSpec((1, N, 64), lambda b: (b, 0, 0)),
            pl.BlockSpec((1, N, 64), lambda b: (b, 0, 0)),
            pl.BlockSpec((1, N, 128), lambda b: (b, 0, 0)),
            pl.BlockSpec((1, N, 256), lambda b: (b, 0, 0)),
            pl.BlockSpec((1, 512), lambda b: (0, 0)),
            pl.BlockSpec((1, 512), lambda b: (0, 0)),
            pl.BlockSpec((512, emb), lambda b: (0, 0)),
            pl.BlockSpec((1, emb), lambda b: (0, 0)),
            pl.BlockSpec((1, emb), lambda b: (0, 0)),
        ],
        out_specs=pl.BlockSpec((1, emb, N), lambda b: (b, 0, 0)),
        out_shape=jax.ShapeDtypeStruct((B, emb, N), jnp.float32),
        compiler_params=pltpu.CompilerParams(
            dimension_semantics=("parallel",),
            vmem_limit_bytes=VMEM_LIMIT),
    )(raw1, raw2, raw3, raw4, sc_cat, sh_cat, w5bf, scale5, shift5)
    return out
